# baseline (speedup 1.0000x reference)
import functools

import jax
import jax.numpy as jnp
from jax import lax
from jax.experimental import pallas as pl
from jax.experimental.pallas import tpu as pltpu
from jax.experimental.pallas import tpu_sc as plsc

K = 64
R = 128
N = 32768
L = 16
NB = N // L
SB = 1024
RB = 256
SAMPLE_STRIDE = 16
SAMPLE_MIN = 32
MASK31 = 0x7FFFFFFF


def _abs_bits(xv):
    return lax.bitcast_convert_type(xv, jnp.int32) & MASK31


def _body(x_hbm, idx_hbm, sgn_hbm, val_hbm,
          row_v, hist, sv, cand, sel, out_u, out_i, stg_val, stg_sgn,
          counts, st):
    info = plsc.get_sparse_core_info()
    nc, ns = info.num_cores, info.num_subcores
    rows_per_w = R // (nc * ns)
    wid = lax.axis_index("s") * nc + lax.axis_index("c")

    lane = lax.iota(jnp.int32, L)
    ones = jnp.ones((L,), jnp.int32)
    zeros_i = jnp.zeros((L,), jnp.int32)

    def pcount(m):
        return plsc.all_reduce_population_count(m)[0]

    def zero_hist(b, carry):
        hist[pl.ds(b * L, L)] = zeros_i
        return carry

    lax.fori_loop(0, SB, zero_hist, 0, unroll=8)

    def s_of(b):
        return jnp.sum(sv[pl.ds(b * L, L)])

    def suffix_scan(nbkt):
        sv[pl.ds(nbkt * L, L)] = zeros_i

        def ss(t, run):
            b = nbkt - 1 - t
            run = run + hist[pl.ds(b * L, L)]
            sv[pl.ds(b * L, L)] = run
            hist[pl.ds(b * L, L)] = zeros_i
            return run

        lax.fori_loop(0, nbkt, ss, zeros_i, unroll=8)

    def bsearch(need, hi_bkt, steps):
        def step(_, lohi):
            lo, hi = lohi
            mid = (lo + hi + 1) // 2
            ge = s_of(mid) >= need
            return jnp.where(ge, mid, lo), jnp.where(ge, hi, mid - 1)

        lo, _ = lax.fori_loop(0, steps, step,
                              (jnp.int32(0), jnp.int32(hi_bkt)))
        return lo

    def split_level(shift, bstar):
        n_c = st[2]
        count_gt = s_of(bstar + 1)
        n_next = s_of(bstar) - count_gt

        def it(t, offs):
            o_s, o_c = offs
            iv = cand[pl.ds(t * L, L)]
            key = _abs_bits(plsc.load_gather(row_v, [iv]))
            digit = (key >> shift) & 255
            valid = (t * L + lane) < n_c
            m_gt = (digit > bstar) & valid
            m_eq = (digit == bstar) & valid
            plsc.store_compressed(sel.at[pl.ds(o_s, L)], iv, mask=m_gt)
            plsc.store_compressed(cand.at[pl.ds(o_c, L)], iv, mask=m_eq)
            return o_s + pcount(m_gt), o_c + pcount(m_eq)

        lax.fori_loop(0, (n_c + L - 1) // L, it, (st[0], jnp.int32(0)))
        st[0] = st[0] + count_gt
        st[1] = st[1] - count_gt
        st[2] = n_next
        plsc.store_scatter(cand, [n_next + lane], zeros_i)

    def do_row(r, carry):
        row = wid * rows_per_w + r
        pltpu.sync_copy(x_hbm.at[row], row_v)

        def hs(t, c):
            bits = lax.bitcast_convert_type(
                row_v[pl.ds(t * SAMPLE_STRIDE * L, L)], jnp.int32)
            addr = ((bits >> 17) & 0x3FF0) | lane
            plsc.addupdate_scatter(hist, [addr], ones)
            return c

        lax.fori_loop(0, NB // SAMPLE_STRIDE, hs, 0, unroll=8)
        suffix_scan(SB)
        b_est = bsearch(jnp.int32(SAMPLE_MIN), SB - 1, 10)
        thr = b_est << 21

        def cntp(t, c):
            key = _abs_bits(row_v[pl.ds(t * L, L)])
            m = key >= thr
            pc = plsc.all_reduce_population_count(m)
            plsc.store_scatter(counts, [jnp.full((L,), t, jnp.int32)], pc,
                               mask=lane == 0)
            return c

        lax.fori_loop(0, NB, cntp, 0, unroll=8)

        lane15 = jnp.full((L,), 15, jnp.int32)

        def prefp(g, carry):
            v = counts[pl.ds(g * L, L)]
            inc = plsc.cumsum(v)
            counts[pl.ds(g * L, L)] = carry + inc - v
            return carry + lax.gather(
                inc, lane15[:, None],
                lax.GatherDimensionNumbers(
                    offset_dims=(), collapsed_slice_dims=(0,),
                    start_index_map=(0,)),
                (1,), mode=lax.GatherScatterMode.PROMISE_IN_BOUNDS)

        total = lax.fori_loop(0, NB // L, prefp, zeros_i)
        n_all = jnp.max(total)

        def scatp(g, c):
            base_vec = counts[pl.ds(g * L, L)]
            for q in range(L):
                t = g * L + q
                key = _abs_bits(row_v[pl.ds(t * L, L)])
                m = key >= thr
                bq = lax.gather(
                    base_vec, jnp.full((L, 1), q, jnp.int32),
                    lax.GatherDimensionNumbers(
                        offset_dims=(), collapsed_slice_dims=(0,),
                        start_index_map=(0,)),
                    (1,), mode=lax.GatherScatterMode.PROMISE_IN_BOUNDS)
                pos = bq + plsc.cumsum(m.astype(jnp.int32)) - 1
                plsc.store_scatter(cand, [pos], t * L + lane, mask=m)
            return c

        lax.fori_loop(0, NB // L, scatp, 0)

        @pl.when(n_all < K)
        def _():
            def fb(t, c):
                cand[pl.ds(t * L, L)] = t * L + lane
                return c

            lax.fori_loop(0, NB, fb, 0, unroll=8)

        n_all = jnp.where(n_all < K, jnp.int32(N), n_all)
        st[0] = jnp.int32(0)
        st[1] = jnp.int32(K)
        st[2] = n_all
        plsc.store_scatter(cand, [n_all + lane], zeros_i)

        for shift in (23, 15, 7, 0):
            @pl.when(st[2] != st[1])
            def _():
                n_c = st[2]

                def hl(t, c):
                    iv = cand[pl.ds(t * L, L)]
                    key = _abs_bits(plsc.load_gather(row_v, [iv]))
                    addr = (((key >> shift) & 255) << 4) | lane
                    m = (t * L + lane) < n_c
                    plsc.addupdate_scatter(hist, [addr], ones, mask=m)
                    return c

                lax.fori_loop(0, (n_c + L - 1) // L, hl, 0)
                suffix_scan(RB)
                split_level(shift, bsearch(st[1], RB - 1, 8))

        sel_off, need = st[0], st[1]

        def tie(t, c):
            iv = cand[pl.ds(t * L, L)]
            p = t * L + lane
            plsc.store_scatter(sel, [sel_off + p], iv, mask=p < need)
            return c

        lax.fori_loop(0, (need + L - 1) // L, tie, 0)

        ks, ivs = [], []
        for q in range(K // L):
            iv = sel[pl.ds(q * L, L)]
            ks.append(_abs_bits(plsc.load_gather(row_v, [iv])))
            ivs.append(iv)

        def pick(j, s_):
            k0, k1, k2, k3, i0, i1, i2, i3 = s_
            m = jnp.max(jnp.maximum(jnp.maximum(k0, k1), jnp.maximum(k2, k3)))
            c0_ = jnp.where(k0 == m, i0, MASK31)
            c1_ = jnp.where(k1 == m, i1, MASK31)
            c2_ = jnp.where(k2 == m, i2, MASK31)
            c3_ = jnp.where(k3 == m, i3, MASK31)
            i = jnp.min(jnp.minimum(jnp.minimum(c0_, c1_), jnp.minimum(c2_, c3_)))
            jv = jnp.full((L,), j, jnp.int32)
            lane0 = lane == 0
            plsc.store_scatter(out_u, [jv], jnp.full((L,), m, jnp.int32), mask=lane0)
            plsc.store_scatter(out_i, [jv], jnp.full((L,), i, jnp.int32), mask=lane0)
            k0 = jnp.where((k0 == m) & (i0 == i), jnp.int32(-1), k0)
            k1 = jnp.where((k1 == m) & (i1 == i), jnp.int32(-1), k1)
            k2 = jnp.where((k2 == m) & (i2 == i), jnp.int32(-1), k2)
            k3 = jnp.where((k3 == m) & (i3 == i), jnp.int32(-1), k3)
            return (k0, k1, k2, k3, i0, i1, i2, i3)

        lax.fori_loop(0, K, pick, tuple(ks) + tuple(ivs))

        for q in range(K // L):
            u = out_u[pl.ds(q * L, L)]
            iv = out_i[pl.ds(q * L, L)]
            stg_val[pl.ds(q * L, L)] = lax.bitcast_convert_type(u, jnp.float32)
            stg_sgn[pl.ds(q * L, L)] = jnp.sign(plsc.load_gather(row_v, [iv]))
        pltpu.sync_copy(out_i, idx_hbm.at[row])
        pltpu.sync_copy(stg_sgn, sgn_hbm.at[row])
        pltpu.sync_copy(stg_val, val_hbm.at[row])
        return carry

    lax.fori_loop(0, rows_per_w, do_row, 0)


@jax.jit
def kernel(x):
    mesh = plsc.VectorSubcoreMesh(core_axis_name="c", subcore_axis_name="s")
    out_type = (
        jax.ShapeDtypeStruct((R, K), jnp.int32),
        jax.ShapeDtypeStruct((R, K), jnp.float32),
        jax.ShapeDtypeStruct((R, K), jnp.float32),
    )
    scratch = [
        pltpu.VMEM((N,), jnp.float32),
        pltpu.VMEM((SB * L,), jnp.int32),
        pltpu.VMEM(((SB + 1) * L,), jnp.int32),
        pltpu.VMEM((N + L,), jnp.int32),
        pltpu.VMEM((K + L,), jnp.int32),
        pltpu.VMEM((K,), jnp.int32),
        pltpu.VMEM((K,), jnp.int32),
        pltpu.VMEM((K,), jnp.float32),
        pltpu.VMEM((K,), jnp.float32),
        pltpu.VMEM((NB,), jnp.int32),
        pltpu.SMEM((4,), jnp.int32),
    ]
    f = pl.kernel(_body, out_type=out_type, mesh=mesh, scratch_types=scratch,
                  compiler_params=pltpu.CompilerParams(needs_layout_passes=False))
    return f(x)


# speedup vs baseline: 1.2462x; 1.2462x over previous
import functools

import jax
import jax.numpy as jnp
from jax import lax
from jax.experimental import pallas as pl
from jax.experimental.pallas import tpu as pltpu
from jax.experimental.pallas import tpu_sc as plsc

K = 64
R = 128
N = 32768
L = 16
NB = N // L
SB = 1024
RB = 256
SAMPLE_STRIDE = 16
SAMPLE_MIN = 32
MASK31 = 0x7FFFFFFF


def _abs_bits(xv):
    return lax.bitcast_convert_type(xv, jnp.int32) & MASK31


def _body(x_hbm, idx_hbm, sgn_hbm, val_hbm,
          row_v, hist, sv, cand, sel, out_u, out_i, stg_val, stg_sgn, st):
    info = plsc.get_sparse_core_info()
    nc, ns = info.num_cores, info.num_subcores
    rows_per_w = R // (nc * ns)
    wid = lax.axis_index("s") * nc + lax.axis_index("c")

    lane = lax.iota(jnp.int32, L)
    ones = jnp.ones((L,), jnp.int32)
    zeros_i = jnp.zeros((L,), jnp.int32)

    def pcount(m):
        return plsc.all_reduce_population_count(m)[0]

    def zero_hist(b, carry):
        hist[pl.ds(b * L, L)] = zeros_i
        return carry

    lax.fori_loop(0, SB, zero_hist, 0, unroll=8)

    def s_of(b):
        return jnp.sum(sv[pl.ds(b * L, L)])

    def suffix_scan(nbkt):
        sv[pl.ds(nbkt * L, L)] = zeros_i

        def ss(t, run):
            b = nbkt - 1 - t
            run = run + hist[pl.ds(b * L, L)]
            sv[pl.ds(b * L, L)] = run
            hist[pl.ds(b * L, L)] = zeros_i
            return run

        lax.fori_loop(0, nbkt, ss, zeros_i, unroll=8)

    def bsearch(need, hi_bkt, steps):
        def step(_, lohi):
            lo, hi = lohi
            mid = (lo + hi + 1) // 2
            ge = s_of(mid) >= need
            return jnp.where(ge, mid, lo), jnp.where(ge, hi, mid - 1)

        lo, _ = lax.fori_loop(0, steps, step,
                              (jnp.int32(0), jnp.int32(hi_bkt)))
        return lo

    def split_level(shift, bstar):
        n_c = st[2]
        count_gt = s_of(bstar + 1)
        n_next = s_of(bstar) - count_gt

        def it(t, offs):
            o_s, o_c = offs
            iv = cand[pl.ds(t * L, L)]
            key = _abs_bits(plsc.load_gather(row_v, [iv]))
            digit = (key >> shift) & 255
            valid = (t * L + lane) < n_c
            m_gt = (digit > bstar) & valid
            m_eq = (digit == bstar) & valid
            plsc.store_compressed(sel.at[pl.ds(o_s, L)], iv, mask=m_gt)
            plsc.store_compressed(cand.at[pl.ds(o_c, L)], iv, mask=m_eq)
            return o_s + pcount(m_gt), o_c + pcount(m_eq)

        lax.fori_loop(0, (n_c + L - 1) // L, it, (st[0], jnp.int32(0)))
        st[0] = st[0] + count_gt
        st[1] = st[1] - count_gt
        st[2] = n_next
        plsc.store_scatter(cand, [n_next + lane], zeros_i)

    def do_row(r, carry):
        row = wid * rows_per_w + r
        pltpu.sync_copy(x_hbm.at[row], row_v)

        def hs(t, c):
            bits = lax.bitcast_convert_type(
                row_v[pl.ds(t * SAMPLE_STRIDE * L, L)], jnp.int32)
            addr = ((bits >> 17) & 0x3FF0) | lane
            plsc.addupdate_scatter(hist, [addr], ones)
            return c

        lax.fori_loop(0, NB // SAMPLE_STRIDE, hs, 0, unroll=8)
        suffix_scan(SB)
        b_est = bsearch(jnp.int32(SAMPLE_MIN), SB - 1, 10)
        thr = b_est << 21

        def p2a(t, o):
            key = _abs_bits(row_v[pl.ds(t * L, L)])
            m = key >= thr
            pos = o + plsc.cumsum(m.astype(jnp.int32)) - 1
            plsc.store_scatter(cand, [pos], t * L + lane, mask=m)
            return o + plsc.all_reduce_population_count(m)

        o_end = lax.fori_loop(0, NB, p2a, zeros_i, unroll=8)
        n_all = jnp.max(o_end)

        @pl.when(n_all < K)
        def _():
            def fb(t, c):
                cand[pl.ds(t * L, L)] = t * L + lane
                return c

            lax.fori_loop(0, NB, fb, 0, unroll=8)

        n_all = jnp.where(n_all < K, jnp.int32(N), n_all)
        st[0] = jnp.int32(0)
        st[1] = jnp.int32(K)
        st[2] = n_all
        plsc.store_scatter(cand, [n_all + lane], zeros_i)

        for shift in (23, 15, 7, 0):
            @pl.when(st[2] != st[1])
            def _():
                n_c = st[2]

                def hl(t, c):
                    iv = cand[pl.ds(t * L, L)]
                    key = _abs_bits(plsc.load_gather(row_v, [iv]))
                    addr = (((key >> shift) & 255) << 4) | lane
                    m = (t * L + lane) < n_c
                    plsc.addupdate_scatter(hist, [addr], ones, mask=m)
                    return c

                lax.fori_loop(0, (n_c + L - 1) // L, hl, 0)
                suffix_scan(RB)
                split_level(shift, bsearch(st[1], RB - 1, 8))

        sel_off, need = st[0], st[1]

        def tie(t, c):
            iv = cand[pl.ds(t * L, L)]
            p = t * L + lane
            plsc.store_scatter(sel, [sel_off + p], iv, mask=p < need)
            return c

        lax.fori_loop(0, (need + L - 1) // L, tie, 0)

        ks, ivs = [], []
        for q in range(K // L):
            iv = sel[pl.ds(q * L, L)]
            ks.append(_abs_bits(plsc.load_gather(row_v, [iv])))
            ivs.append(iv)

        def pick(j, s_):
            k0, k1, k2, k3, i0, i1, i2, i3 = s_
            m = jnp.max(jnp.maximum(jnp.maximum(k0, k1), jnp.maximum(k2, k3)))
            c0_ = jnp.where(k0 == m, i0, MASK31)
            c1_ = jnp.where(k1 == m, i1, MASK31)
            c2_ = jnp.where(k2 == m, i2, MASK31)
            c3_ = jnp.where(k3 == m, i3, MASK31)
            i = jnp.min(jnp.minimum(jnp.minimum(c0_, c1_), jnp.minimum(c2_, c3_)))
            jv = jnp.full((L,), j, jnp.int32)
            lane0 = lane == 0
            plsc.store_scatter(out_u, [jv], jnp.full((L,), m, jnp.int32), mask=lane0)
            plsc.store_scatter(out_i, [jv], jnp.full((L,), i, jnp.int32), mask=lane0)
            k0 = jnp.where((k0 == m) & (i0 == i), jnp.int32(-1), k0)
            k1 = jnp.where((k1 == m) & (i1 == i), jnp.int32(-1), k1)
            k2 = jnp.where((k2 == m) & (i2 == i), jnp.int32(-1), k2)
            k3 = jnp.where((k3 == m) & (i3 == i), jnp.int32(-1), k3)
            return (k0, k1, k2, k3, i0, i1, i2, i3)

        lax.fori_loop(0, K, pick, tuple(ks) + tuple(ivs))

        for q in range(K // L):
            u = out_u[pl.ds(q * L, L)]
            iv = out_i[pl.ds(q * L, L)]
            stg_val[pl.ds(q * L, L)] = lax.bitcast_convert_type(u, jnp.float32)
            stg_sgn[pl.ds(q * L, L)] = jnp.sign(plsc.load_gather(row_v, [iv]))
        pltpu.sync_copy(out_i, idx_hbm.at[row])
        pltpu.sync_copy(stg_sgn, sgn_hbm.at[row])
        pltpu.sync_copy(stg_val, val_hbm.at[row])
        return carry

    lax.fori_loop(0, rows_per_w, do_row, 0)


@jax.jit
def kernel(x):
    mesh = plsc.VectorSubcoreMesh(core_axis_name="c", subcore_axis_name="s")
    out_type = (
        jax.ShapeDtypeStruct((R, K), jnp.int32),
        jax.ShapeDtypeStruct((R, K), jnp.float32),
        jax.ShapeDtypeStruct((R, K), jnp.float32),
    )
    scratch = [
        pltpu.VMEM((N,), jnp.float32),
        pltpu.VMEM((SB * L,), jnp.int32),
        pltpu.VMEM(((SB + 1) * L,), jnp.int32),
        pltpu.VMEM((N + L,), jnp.int32),
        pltpu.VMEM((K + L,), jnp.int32),
        pltpu.VMEM((K,), jnp.int32),
        pltpu.VMEM((K,), jnp.int32),
        pltpu.VMEM((K,), jnp.float32),
        pltpu.VMEM((K,), jnp.float32),
        pltpu.SMEM((4,), jnp.int32),
    ]
    f = pl.kernel(_body, out_type=out_type, mesh=mesh, scratch_types=scratch,
                  compiler_params=pltpu.CompilerParams(needs_layout_passes=False))
    return f(x)


# speedup vs baseline: 1.4589x; 1.1707x over previous
import functools

import jax
import jax.numpy as jnp
from jax import lax
from jax.experimental import pallas as pl
from jax.experimental.pallas import tpu as pltpu
from jax.experimental.pallas import tpu_sc as plsc

K = 64
R = 128
N = 32768
L = 16
NB = N // L
SB = 1024
RB = 256
SAMPLE_STRIDE = 16
SAMPLE_MIN = 32
MASK31 = 0x7FFFFFFF


def _abs_bits(xv):
    return lax.bitcast_convert_type(xv, jnp.int32) & MASK31


def _body(x_hbm, idx_hbm, sgn_hbm, val_hbm,
          row_v, hist, sv, cand, sel, out_u, out_i, stg_val, stg_sgn, st):
    info = plsc.get_sparse_core_info()
    nc, ns = info.num_cores, info.num_subcores
    rows_per_w = R // (nc * ns)
    wid = lax.axis_index("s") * nc + lax.axis_index("c")

    lane = lax.iota(jnp.int32, L)
    ones = jnp.ones((L,), jnp.int32)
    zeros_i = jnp.zeros((L,), jnp.int32)

    def pcount(m):
        return plsc.all_reduce_population_count(m)[0]

    def zero_hist(b, carry):
        hist[pl.ds(b * L, L)] = zeros_i
        return carry

    lax.fori_loop(0, SB, zero_hist, 0, unroll=8)

    def s_of(b):
        return jnp.sum(sv[pl.ds(b * L, L)])

    def suffix_scan(nbkt):
        sv[pl.ds(nbkt * L, L)] = zeros_i

        def ss(t, run):
            b = nbkt - 1 - t
            run = run + hist[pl.ds(b * L, L)]
            sv[pl.ds(b * L, L)] = run
            hist[pl.ds(b * L, L)] = zeros_i
            return run

        lax.fori_loop(0, nbkt, ss, zeros_i, unroll=8)

    def bsearch(need, hi_bkt, steps):
        def step(_, lohi):
            lo, hi = lohi
            mid = (lo + hi + 1) // 2
            ge = s_of(mid) >= need
            return jnp.where(ge, mid, lo), jnp.where(ge, hi, mid - 1)

        lo, _ = lax.fori_loop(0, steps, step,
                              (jnp.int32(0), jnp.int32(hi_bkt)))
        return lo

    def split_level(shift, bstar):
        n_c = st[2]
        count_gt = s_of(bstar + 1)
        n_next = s_of(bstar) - count_gt

        def it(t, offs):
            o_s, o_c = offs
            iv = cand[pl.ds(t * L, L)]
            key = _abs_bits(plsc.load_gather(row_v, [iv]))
            digit = (key >> shift) & 255
            valid = (t * L + lane) < n_c
            m_gt = (digit > bstar) & valid
            m_eq = (digit == bstar) & valid
            plsc.store_compressed(sel.at[pl.ds(o_s, L)], iv, mask=m_gt)
            plsc.store_compressed(cand.at[pl.ds(o_c, L)], iv, mask=m_eq)
            return o_s + pcount(m_gt), o_c + pcount(m_eq)

        lax.fori_loop(0, (n_c + L - 1) // L, it, (st[0], jnp.int32(0)))
        st[0] = st[0] + count_gt
        st[1] = st[1] - count_gt
        st[2] = n_next
        plsc.store_scatter(cand, [n_next + lane], zeros_i)

    def do_row(r, carry):
        row = wid * rows_per_w + r
        pltpu.sync_copy(x_hbm.at[row], row_v)

        def hs(t, c):
            bits = lax.bitcast_convert_type(
                row_v[pl.ds(t * SAMPLE_STRIDE * L, L)], jnp.int32)
            addr = ((bits >> 17) & 0x3FF0) | lane
            plsc.addupdate_scatter(hist, [addr], ones)
            return c

        lax.fori_loop(0, NB // SAMPLE_STRIDE, hs, 0, unroll=8)
        suffix_scan(SB)
        b_est = bsearch(jnp.int32(SAMPLE_MIN), SB - 1, 10)
        thr = b_est << 21

        lane_base = lane * NB

        def p2a(t, off):
            key = _abs_bits(row_v[pl.ds(t * L, L)])
            m = key >= thr
            plsc.store_scatter(cand, [lane_base + off], t * L + lane, mask=m)
            return off + m.astype(jnp.int32)

        off_vec = lax.fori_loop(0, NB, p2a, zeros_i, unroll=8)
        n_all = jnp.sum(off_vec)
        bad = (n_all < K) | (n_all > NB)

        @pl.when(bad)
        def _():
            def fb(t, c):
                cand[pl.ds(t * L, L)] = t * L + lane
                return c

            lax.fori_loop(0, NB, fb, 0, unroll=8)

        @pl.when(jnp.logical_not(bad))
        def _():
            o = jnp.int32(0)
            for l in range(L):
                cnt = off_vec[l]

                def cp(j, c, l=l, cnt=cnt, o=o):
                    v = cand[pl.ds(l * NB + j * L, L)]
                    plsc.store_compressed(cand.at[pl.ds(o + j * L, L)], v,
                                          mask=(j * L + lane) < cnt)
                    return c

                lax.fori_loop(0, (cnt + L - 1) // L, cp, 0)
                o = o + cnt

        n_all = jnp.where(bad, jnp.int32(N), n_all)
        st[0] = jnp.int32(0)
        st[1] = jnp.int32(K)
        st[2] = n_all
        plsc.store_scatter(cand, [n_all + lane], zeros_i)

        for shift in (23, 15, 7, 0):
            @pl.when(st[2] != st[1])
            def _():
                n_c = st[2]

                def hl(t, c):
                    iv = cand[pl.ds(t * L, L)]
                    key = _abs_bits(plsc.load_gather(row_v, [iv]))
                    addr = (((key >> shift) & 255) << 4) | lane
                    m = (t * L + lane) < n_c
                    plsc.addupdate_scatter(hist, [addr], ones, mask=m)
                    return c

                lax.fori_loop(0, (n_c + L - 1) // L, hl, 0)
                suffix_scan(RB)
                split_level(shift, bsearch(st[1], RB - 1, 8))

        sel_off, need = st[0], st[1]
        n_c = st[2]
        nt = (n_c + L - 1) // L

        def tie_one(j, c):
            def mn(t, cur):
                iv = cand[pl.ds(t * L, L)]
                valid = (t * L + lane) < n_c
                return jnp.minimum(cur, jnp.where(valid, iv, MASK31))

            mvec = lax.fori_loop(0, nt, mn,
                                 jnp.full((L,), MASK31, jnp.int32))
            mi = jnp.min(mvec)
            plsc.store_scatter(sel, [jnp.full((L,), sel_off + j, jnp.int32)],
                               jnp.full((L,), mi, jnp.int32), mask=lane == 0)

            def rm(t, c2):
                iv = cand[pl.ds(t * L, L)]
                cand[pl.ds(t * L, L)] = jnp.where(iv == mi, MASK31, iv)
                return c2

            lax.fori_loop(0, nt, rm, 0)
            return c

        lax.fori_loop(0, need, tie_one, 0)

        ks, ivs = [], []
        for q in range(K // L):
            iv = sel[pl.ds(q * L, L)]
            ks.append(_abs_bits(plsc.load_gather(row_v, [iv])))
            ivs.append(iv)

        def pick(j, s_):
            k0, k1, k2, k3, i0, i1, i2, i3 = s_
            m = jnp.max(jnp.maximum(jnp.maximum(k0, k1), jnp.maximum(k2, k3)))
            c0_ = jnp.where(k0 == m, i0, MASK31)
            c1_ = jnp.where(k1 == m, i1, MASK31)
            c2_ = jnp.where(k2 == m, i2, MASK31)
            c3_ = jnp.where(k3 == m, i3, MASK31)
            i = jnp.min(jnp.minimum(jnp.minimum(c0_, c1_), jnp.minimum(c2_, c3_)))
            jv = jnp.full((L,), j, jnp.int32)
            lane0 = lane == 0
            plsc.store_scatter(out_u, [jv], jnp.full((L,), m, jnp.int32), mask=lane0)
            plsc.store_scatter(out_i, [jv], jnp.full((L,), i, jnp.int32), mask=lane0)
            k0 = jnp.where((k0 == m) & (i0 == i), jnp.int32(-1), k0)
            k1 = jnp.where((k1 == m) & (i1 == i), jnp.int32(-1), k1)
            k2 = jnp.where((k2 == m) & (i2 == i), jnp.int32(-1), k2)
            k3 = jnp.where((k3 == m) & (i3 == i), jnp.int32(-1), k3)
            return (k0, k1, k2, k3, i0, i1, i2, i3)

        lax.fori_loop(0, K, pick, tuple(ks) + tuple(ivs))

        for q in range(K // L):
            u = out_u[pl.ds(q * L, L)]
            iv = out_i[pl.ds(q * L, L)]
            stg_val[pl.ds(q * L, L)] = lax.bitcast_convert_type(u, jnp.float32)
            stg_sgn[pl.ds(q * L, L)] = jnp.sign(plsc.load_gather(row_v, [iv]))
        pltpu.sync_copy(out_i, idx_hbm.at[row])
        pltpu.sync_copy(stg_sgn, sgn_hbm.at[row])
        pltpu.sync_copy(stg_val, val_hbm.at[row])
        return carry

    lax.fori_loop(0, rows_per_w, do_row, 0)


@jax.jit
def kernel(x):
    mesh = plsc.VectorSubcoreMesh(core_axis_name="c", subcore_axis_name="s")
    out_type = (
        jax.ShapeDtypeStruct((R, K), jnp.int32),
        jax.ShapeDtypeStruct((R, K), jnp.float32),
        jax.ShapeDtypeStruct((R, K), jnp.float32),
    )
    scratch = [
        pltpu.VMEM((N,), jnp.float32),
        pltpu.VMEM((SB * L,), jnp.int32),
        pltpu.VMEM(((SB + 1) * L,), jnp.int32),
        pltpu.VMEM((N + L,), jnp.int32),
        pltpu.VMEM((K + L,), jnp.int32),
        pltpu.VMEM((K,), jnp.int32),
        pltpu.VMEM((K,), jnp.int32),
        pltpu.VMEM((K,), jnp.float32),
        pltpu.VMEM((K,), jnp.float32),
        pltpu.SMEM((4,), jnp.int32),
    ]
    f = pl.kernel(_body, out_type=out_type, mesh=mesh, scratch_types=scratch,
                  compiler_params=pltpu.CompilerParams(needs_layout_passes=False))
    return f(x)
